# baseline (speedup 1.0000x reference)
import jax
import jax.numpy as jnp
from jax.experimental import pallas as pl
from jax.experimental.pallas import tpu as pltpu

_CHUNK = 8192
_NBUF = 1


def _dma_body(*refs):
    w_hbm, o_hbm = refs[0], refs[1]
    bufs = refs[2 : 2 + _NBUF]
    rsem, wsem = refs[2 + _NBUF], refs[3 + _NBUF]
    batch, seq_len, _ = o_hbm.shape
    n_chunks = seq_len // _CHUNK

    reads = [
        pltpu.async_copy(
            w_hbm.at[pl.ds(c * _CHUNK, _CHUNK), :], bufs[c], rsem.at[c]
        )
        for c in range(n_chunks)
    ]
    writes = []
    for c in range(n_chunks):
        reads[c].wait()
        writes.extend(
            pltpu.async_copy(
                bufs[c], o_hbm.at[b, pl.ds(c * _CHUNK, _CHUNK), :], wsem.at[c]
            )
            for b in range(batch)
        )
    for h in writes:
        h.wait()


def kernel(x, pos_weight):
    batch, seq_len = x.shape
    embed_dim = pos_weight.shape[1]
    assert seq_len == _CHUNK * _NBUF

    out = pl.pallas_call(
        _dma_body,
        in_specs=[pl.BlockSpec(memory_space=pl.ANY)],
        out_specs=pl.BlockSpec(memory_space=pl.ANY),
        out_shape=jax.ShapeDtypeStruct((batch, seq_len, embed_dim), pos_weight.dtype),
        scratch_shapes=[pltpu.VMEM((_CHUNK, embed_dim), pos_weight.dtype)] * _NBUF
        + [pltpu.SemaphoreType.DMA((_NBUF,)), pltpu.SemaphoreType.DMA((_NBUF,))],
    )(pos_weight)
    return out


# speedup vs baseline: 1.0545x; 1.0545x over previous
import jax
import jax.numpy as jnp
from jax.experimental import pallas as pl
from jax.experimental.pallas import tpu as pltpu

_CHUNK = 2048
_NBUF = 4


def _dma_body(*refs):
    w_hbm, o_hbm = refs[0], refs[1]
    bufs = refs[2 : 2 + _NBUF]
    rsem, wsem = refs[2 + _NBUF], refs[3 + _NBUF]
    batch, seq_len, _ = o_hbm.shape
    n_chunks = seq_len // _CHUNK

    reads = [
        pltpu.async_copy(
            w_hbm.at[pl.ds(c * _CHUNK, _CHUNK), :], bufs[c], rsem.at[c]
        )
        for c in range(n_chunks)
    ]
    writes = []
    for c in range(n_chunks):
        reads[c].wait()
        writes.extend(
            pltpu.async_copy(
                bufs[c], o_hbm.at[b, pl.ds(c * _CHUNK, _CHUNK), :], wsem.at[c]
            )
            for b in range(batch)
        )
    for h in writes:
        h.wait()


def kernel(x, pos_weight):
    batch, seq_len = x.shape
    embed_dim = pos_weight.shape[1]
    assert seq_len == _CHUNK * _NBUF

    out = pl.pallas_call(
        _dma_body,
        in_specs=[pl.BlockSpec(memory_space=pl.ANY)],
        out_specs=pl.BlockSpec(memory_space=pl.ANY),
        out_shape=jax.ShapeDtypeStruct((batch, seq_len, embed_dim), pos_weight.dtype),
        scratch_shapes=[pltpu.VMEM((_CHUNK, embed_dim), pos_weight.dtype)] * _NBUF
        + [pltpu.SemaphoreType.DMA((_NBUF,)), pltpu.SemaphoreType.DMA((_NBUF,))],
    )(pos_weight)
    return out
